# baseline (speedup 1.0000x reference)
import functools

import jax
import jax.numpy as jnp
from jax import lax
from jax.experimental import pallas as pl
from jax.experimental.pallas import tpu as pltpu
from jax.experimental.pallas import tpu_sc as plsc

N_NODES = 10000
N_EDGES = 320000
D_FEAT = 128
D_EMB = 128
D_TARGET = 32
N_GRAPHS = 64

NC = 2
NS = 16
NW = NC * NS
E_PER_W = N_EDGES // NW
K = 128
NSTAGE = 2
CH_STAGE = 40
CH_LOAD = 48
NCHUNK_H = 88
E_PAD = NCHUNK_H * K - E_PER_W
N_ACC = N_NODES + 8
ROWS_PER_S = 624
TAIL_BASE = ROWS_PER_S * NS
TAIL_ROWS = N_ACC - TAIL_BASE


def _sc_body(x_h, src_h, dst_h, zer_h, zerd_h, one_h,
             agg_o, deg_o,
             idx_s, idx_d, rows0, rows1, ones_v, agg_sh, deg_sh,
             semg0, semg1):
    cid = lax.axis_index("c")
    sid = lax.axis_index("s")
    wid = cid * NS + sid

    pltpu.sync_copy(zer_h.at[pl.ds(sid * ROWS_PER_S, ROWS_PER_S)],
                    agg_sh.at[pl.ds(sid * ROWS_PER_S, ROWS_PER_S)])

    @pl.when(sid == NS - 1)
    def _zero_tail():
        pltpu.sync_copy(zer_h.at[pl.ds(TAIL_BASE, TAIL_ROWS)],
                        agg_sh.at[pl.ds(TAIL_BASE, TAIL_ROWS)])

    @pl.when(sid == 0)
    def _zero_deg():
        pltpu.sync_copy(zerd_h, deg_sh)

    pltpu.sync_copy(one_h, ones_v)
    plsc.subcore_barrier()

    def gather(j, buf, sem):
        pltpu.async_copy(x_h.at[idx_s.at[j]], buf, sem)

    def wait_gather(j, buf, sem):
        pltpu.make_async_copy(x_h.at[idx_s.at[j]], buf, sem).wait()

    for stage in range(NSTAGE):
        base = stage * CH_STAGE
        pltpu.sync_copy(src_h.at[wid, pl.ds(base, CH_LOAD)], idx_s)
        pltpu.sync_copy(dst_h.at[wid, pl.ds(base, CH_LOAD)], idx_d)
        gather(0, rows0, semg0)

        def step(j2, carry):
            a = 2 * j2
            wait_gather(a, rows0, semg0)
            gather(a + 1, rows1, semg1)
            pltpu.sync_copy(rows0, agg_sh.at[idx_d.at[a]], add=True)
            pltpu.sync_copy(ones_v, deg_sh.at[idx_d.at[a]], add=True)
            wait_gather(a + 1, rows1, semg1)
            gather(a + 2, rows0, semg0)
            pltpu.sync_copy(rows1, agg_sh.at[idx_d.at[a + 1]], add=True)
            pltpu.sync_copy(ones_v, deg_sh.at[idx_d.at[a + 1]], add=True)
            return carry

        lax.fori_loop(0, CH_STAGE // 2, step, 0)
        wait_gather(CH_STAGE, rows0, semg0)

    plsc.subcore_barrier()

    pltpu.sync_copy(agg_sh.at[pl.ds(sid * ROWS_PER_S, ROWS_PER_S)],
                    agg_o.at[cid, pl.ds(sid * ROWS_PER_S, ROWS_PER_S)])

    @pl.when(sid == NS - 1)
    def _out_tail():
        pltpu.sync_copy(agg_sh.at[pl.ds(TAIL_BASE, TAIL_ROWS)],
                        agg_o.at[cid, pl.ds(TAIL_BASE, TAIL_ROWS)])

    @pl.when(sid == 0)
    def _out_deg():
        pltpu.sync_copy(deg_sh, deg_o.at[cid, 0])


_sc_agg = functools.partial(
    pl.kernel,
    out_type=(
        jax.ShapeDtypeStruct((NC, N_ACC, D_FEAT), jnp.float32),
        jax.ShapeDtypeStruct((NC, 1, N_ACC), jnp.float32),
    ),
    mesh=plsc.VectorSubcoreMesh(core_axis_name="c", subcore_axis_name="s"),
    scratch_types=[
        pltpu.VMEM((CH_LOAD, K), jnp.int32),
        pltpu.VMEM((CH_LOAD, K), jnp.int32),
        pltpu.VMEM((K, D_FEAT), jnp.float32),
        pltpu.VMEM((K, D_FEAT), jnp.float32),
        pltpu.VMEM((K,), jnp.float32),
        pltpu.VMEM_SHARED((N_ACC, D_FEAT), jnp.float32),
        pltpu.VMEM_SHARED((N_ACC,), jnp.float32),
        pltpu.SemaphoreType.DMA,
        pltpu.SemaphoreType.DMA,
    ],
)(_sc_body)


def _tc_body(agg_ref, d0_ref, d1_ref, x_ref, batch_ref, wl_ref, bl_ref,
             wr_ref, wro_ref, bro_ref, out_ref, emb_ref):
    agg = agg_ref[0, :N_NODES, :] + agg_ref[1, :N_NODES, :]
    deg = d0_ref[...] + d1_ref[...]
    mean = agg / jnp.maximum(deg, 1.0)
    f32 = jnp.float32
    emb = (lax.dot_general(mean, wl_ref[...], (((1,), (1,)), ((), ())),
                           preferred_element_type=f32)
           + lax.dot_general(x_ref[...], wr_ref[...], (((1,), (1,)), ((), ())),
                             preferred_element_type=f32)
           + bl_ref[...])
    emb_ref[...] = emb
    gids = lax.broadcasted_iota(jnp.int32, (N_GRAPHS, N_NODES), 0)
    onehot = (gids == batch_ref[...]).astype(f32)
    psum = lax.dot_general(onehot, emb, (((1,), (0,)), ((), ())),
                           preferred_element_type=f32)
    cnt = jnp.sum(onehot, axis=1, keepdims=True)
    pooled = psum / jnp.maximum(cnt, 1.0)
    out_ref[...] = (lax.dot_general(pooled, wro_ref[...], (((1,), (1,)), ((), ())),
                                    preferred_element_type=f32)
                    + bro_ref[...])


def kernel(x, edge_index, batch, W_l, b_l, W_r, W_ro, b_ro):
    src_w = edge_index[0].reshape(NW, E_PER_W)
    dst_w = edge_index[1].reshape(NW, E_PER_W)
    src = jnp.concatenate(
        [src_w, jnp.zeros((NW, E_PAD), jnp.int32)], axis=1).reshape(NW, NCHUNK_H, K)
    dst = jnp.concatenate(
        [dst_w, jnp.full((NW, E_PAD), N_NODES, jnp.int32)], axis=1).reshape(NW, NCHUNK_H, K)
    zeros2d = jnp.zeros((N_ACC, D_FEAT), jnp.float32)
    zerosd = jnp.zeros((N_ACC,), jnp.float32)
    ones_k = jnp.ones((K,), jnp.float32)

    agg2, deg2 = _sc_agg(x, src, dst, zeros2d, zerosd, ones_k)
    d0 = deg2[0, 0, :N_NODES].reshape(N_NODES, 1)
    d1 = deg2[1, 0, :N_NODES].reshape(N_NODES, 1)

    out, emb = pl.pallas_call(
        _tc_body,
        out_shape=(
            jax.ShapeDtypeStruct((N_GRAPHS, D_TARGET), jnp.float32),
            jax.ShapeDtypeStruct((N_NODES, D_EMB), jnp.float32),
        ),
    )(agg2, d0, d1, x, batch.reshape(1, N_NODES),
      W_l, b_l.reshape(1, D_EMB), W_r, W_ro, b_ro.reshape(1, D_TARGET))
    return (out, emb)


# speedup vs baseline: 1.7115x; 1.7115x over previous
import functools

import jax
import jax.numpy as jnp
from jax import lax
from jax.experimental import pallas as pl
from jax.experimental.pallas import tpu as pltpu
from jax.experimental.pallas import tpu_sc as plsc

N_NODES = 10000
N_EDGES = 320000
D_FEAT = 128
D_EMB = 128
D_TARGET = 32
N_GRAPHS = 64

NC = 2
NS = 16
NW = NC * NS
E_PER_W = N_EDGES // NW
K = 128
NCHUNK = -(-E_PER_W // K)
E_PAD = NCHUNK * K - E_PER_W
N_ACC = N_NODES + 8
ROWS_PER_S = 624
TAIL_BASE = ROWS_PER_S * NS
TAIL_ROWS = N_ACC - TAIL_BASE


def _sc_body(x_h, src_h, dst_h, zer_h, zerd_h, one_h,
             agg_o, deg_o,
             idx_s, idx_d, rows, ones_v, agg_sh, deg_sh, sem):
    cid = lax.axis_index("c")
    sid = lax.axis_index("s")
    wid = cid * NS + sid

    pltpu.sync_copy(zer_h.at[pl.ds(sid * ROWS_PER_S, ROWS_PER_S)],
                    agg_sh.at[pl.ds(sid * ROWS_PER_S, ROWS_PER_S)])

    @pl.when(sid == NS - 1)
    def _zero_tail():
        pltpu.sync_copy(zer_h.at[pl.ds(TAIL_BASE, TAIL_ROWS)],
                        agg_sh.at[pl.ds(TAIL_BASE, TAIL_ROWS)])

    @pl.when(sid == 0)
    def _zero_deg():
        pltpu.sync_copy(zerd_h, deg_sh)

    pltpu.sync_copy(src_h.at[wid], idx_s)
    pltpu.sync_copy(dst_h.at[wid], idx_d)
    pltpu.sync_copy(one_h, ones_v)
    plsc.subcore_barrier()

    def chunk(j, carry):
        pltpu.async_copy(x_h.at[idx_s.at[j]], rows, sem).wait()
        pltpu.sync_copy(rows, agg_sh.at[idx_d.at[j]], add=True)
        pltpu.sync_copy(ones_v, deg_sh.at[idx_d.at[j]], add=True)
        return carry

    lax.fori_loop(0, NCHUNK, chunk, 0)
    plsc.subcore_barrier()

    pltpu.sync_copy(agg_sh.at[pl.ds(sid * ROWS_PER_S, ROWS_PER_S)],
                    agg_o.at[cid, pl.ds(sid * ROWS_PER_S, ROWS_PER_S)])

    @pl.when(sid == NS - 1)
    def _out_tail():
        pltpu.sync_copy(agg_sh.at[pl.ds(TAIL_BASE, TAIL_ROWS)],
                        agg_o.at[cid, pl.ds(TAIL_BASE, TAIL_ROWS)])

    @pl.when(sid == 0)
    def _out_deg():
        pltpu.sync_copy(deg_sh, deg_o.at[cid, 0])


_sc_agg = functools.partial(
    pl.kernel,
    out_type=(
        jax.ShapeDtypeStruct((NC, N_ACC, D_FEAT), jnp.float32),
        jax.ShapeDtypeStruct((NC, 1, N_ACC), jnp.float32),
    ),
    mesh=plsc.VectorSubcoreMesh(core_axis_name="c", subcore_axis_name="s"),
    scratch_types=[
        pltpu.VMEM((NCHUNK, K), jnp.int32),
        pltpu.VMEM((NCHUNK, K), jnp.int32),
        pltpu.VMEM((K, D_FEAT), jnp.float32),
        pltpu.VMEM((K,), jnp.float32),
        pltpu.VMEM_SHARED((N_ACC, D_FEAT), jnp.float32),
        pltpu.VMEM_SHARED((N_ACC,), jnp.float32),
        pltpu.SemaphoreType.DMA,
    ],
)(_sc_body)


def _xr_body(x_ref, wr_ref, bl_ref, xr_ref):
    xr_ref[...] = (lax.dot_general(x_ref[...], wr_ref[...],
                                   (((1,), (1,)), ((), ())),
                                   preferred_element_type=jnp.float32)
                   + bl_ref[...])


def _tc_body(agg_ref, d0_ref, d1_ref, xr_ref, batch_ref, wl_ref,
             wro_ref, bro_ref, out_ref, emb_ref):
    agg = agg_ref[0, :N_NODES, :] + agg_ref[1, :N_NODES, :]
    deg = d0_ref[...] + d1_ref[...]
    mean = agg / jnp.maximum(deg, 1.0)
    f32 = jnp.float32
    emb = (lax.dot_general(mean, wl_ref[...], (((1,), (1,)), ((), ())),
                           preferred_element_type=f32)
           + xr_ref[...])
    emb_ref[...] = emb
    gids = lax.broadcasted_iota(jnp.int32, (N_GRAPHS, N_NODES), 0)
    onehot = (gids == batch_ref[...]).astype(f32)
    psum = lax.dot_general(onehot, emb, (((1,), (0,)), ((), ())),
                           preferred_element_type=f32)
    cnt = jnp.sum(onehot, axis=1, keepdims=True)
    pooled = psum / jnp.maximum(cnt, 1.0)
    out_ref[...] = (lax.dot_general(pooled, wro_ref[...], (((1,), (1,)), ((), ())),
                                    preferred_element_type=f32)
                    + bro_ref[...])


def kernel(x, edge_index, batch, W_l, b_l, W_r, W_ro, b_ro):
    src_w = edge_index[0].reshape(NW, E_PER_W)
    dst_w = edge_index[1].reshape(NW, E_PER_W)
    src = jnp.concatenate(
        [src_w, jnp.zeros((NW, E_PAD), jnp.int32)], axis=1).reshape(NW, NCHUNK, K)
    dst = jnp.concatenate(
        [dst_w, jnp.full((NW, E_PAD), N_NODES, jnp.int32)], axis=1).reshape(NW, NCHUNK, K)
    zeros2d = jnp.zeros((N_ACC, D_FEAT), jnp.float32)
    zerosd = jnp.zeros((N_ACC,), jnp.float32)
    ones_k = jnp.ones((K,), jnp.float32)

    xr = pl.pallas_call(
        _xr_body,
        out_shape=jax.ShapeDtypeStruct((N_NODES, D_EMB), jnp.float32),
    )(x, W_r, b_l.reshape(1, D_EMB))

    agg2, deg2 = _sc_agg(x, src, dst, zeros2d, zerosd, ones_k)
    d0 = deg2[0, 0, :N_NODES].reshape(N_NODES, 1)
    d1 = deg2[1, 0, :N_NODES].reshape(N_NODES, 1)

    out, emb = pl.pallas_call(
        _tc_body,
        out_shape=(
            jax.ShapeDtypeStruct((N_GRAPHS, D_TARGET), jnp.float32),
            jax.ShapeDtypeStruct((N_NODES, D_EMB), jnp.float32),
        ),
    )(agg2, d0, d1, xr, batch.reshape(1, N_NODES),
      W_l, W_ro, b_ro.reshape(1, D_TARGET))
    return (out, emb)
